# Initial kernel scaffold; baseline (speedup 1.0000x reference)
import functools
import math

import jax
import jax.numpy as jnp
from jax.experimental import pallas as pl
from jax.experimental.pallas import tpu as pltpu

R_MAX = 5.0
_INV_SQRT_AVG = 1.0 / math.sqrt(32.0)

_BE = 2048
_BN = 2048

_PREC = jax.lax.Precision.HIGHEST


def _dot(a, b):
    return jax.lax.dot_general(a, b, (((1,), (0,)), ((), ())),
                               preferred_element_type=jnp.float32,
                               precision=_PREC)


def _silu(x):
    return x * jax.nn.sigmoid(x)


def _geom_body(ps_ref, pr_ref, u_ref, rb_ref):
    ps = ps_ref[...]
    pr = pr_ref[...]
    rel = (pr - ps) * (1.0 / R_MAX)
    r2 = jnp.sum(rel * rel, axis=1, keepdims=True)
    r = jnp.sqrt(r2)
    r_safe = jnp.maximum(r, 1e-6)
    u_ref[...] = rel / r_safe
    n = jnp.arange(1, 9, dtype=jnp.float32)[None, :]
    rb = math.sqrt(2.0) * jnp.sin(jnp.pi * r_safe * n) / r_safe
    rc = jnp.clip(r, 0.0, 1.0)
    rc2 = rc * rc
    rc4 = rc2 * rc2
    rc6 = rc4 * rc2
    rc7 = rc6 * rc
    rc8 = rc4 * rc4
    env = 1.0 - 28.0 * rc6 + 48.0 * rc7 - 21.0 * rc8
    rb_ref[...] = rb * env


def _geometry(pos_s, pos_r):
    Ep = pos_s.shape[0]
    grid = (Ep // _BE,)
    bs = lambda w: pl.BlockSpec((_BE, w), lambda i: (i, 0))
    return pl.pallas_call(
        _geom_body,
        grid=grid,
        in_specs=[bs(4), bs(4)],
        out_specs=[bs(4), bs(8)],
        out_shape=[jax.ShapeDtypeStruct((Ep, 4), jnp.float32),
                   jax.ShapeDtypeStruct((Ep, 8), jnp.float32)],
    )(pos_s, pos_r)


def _embed_body(spec_ref, emb_ref, out_ref):
    spec = spec_ref[...]
    ns = emb_ref.shape[0]
    oh = (spec == jax.lax.broadcasted_iota(jnp.int32, (spec.shape[0], ns), 1))
    out_ref[...] = _dot(oh.astype(jnp.float32), emb_ref[...])


def _embed(spec_p, emb):
    Np = spec_p.shape[0]
    C = emb.shape[1]
    return pl.pallas_call(
        _embed_body,
        grid=(Np // _BN,),
        in_specs=[pl.BlockSpec((_BN, 1), lambda i: (i, 0)),
                  pl.BlockSpec(emb.shape, lambda i: (0, 0))],
        out_specs=pl.BlockSpec((_BN, C), lambda i: (i, 0)),
        out_shape=jax.ShapeDtypeStruct((Np, C), jnp.float32),
    )(spec_p, emb)


def _edge_body(has_v, rb_ref, u_ref, feat_ref, w1_ref, b1_ref, w2_ref, b2_ref,
               wo_ref, ms_ref, mx_ref, my_ref, mz_ref):
    rb = rb_ref[...]
    h = _silu(_dot(rb, w1_ref[...]) + b1_ref[...])
    h = _silu(_dot(h, w2_ref[...]) + b2_ref[...])
    w = _dot(h, wo_ref[...])
    C = ms_ref.shape[1]
    w0 = w[:, 0 * C:1 * C]
    w1 = w[:, 1 * C:2 * C]
    w2 = w[:, 2 * C:3 * C]
    w3 = w[:, 3 * C:4 * C]
    w4 = w[:, 4 * C:5 * C]
    w5 = w[:, 5 * C:6 * C]
    u = u_ref[...]
    ux = u[:, 0:1]
    uy = u[:, 1:2]
    uz = u[:, 2:3]
    s_e = feat_ref[:, 0:C]
    if has_v:
        vx = feat_ref[:, C:2 * C]
        vy = feat_ref[:, 2 * C:3 * C]
        vz = feat_ref[:, 3 * C:4 * C]
        dot_vY = vx * ux + vy * uy + vz * uz
        cx = vy * uz - vz * uy
        cy = vz * ux - vx * uz
        cz = vx * uy - vy * ux
        third = 1.0 / 3.0
        ms_ref[...] = w0 * s_e + w3 * dot_vY
        su = w1 * s_e
        mx_ref[...] = su * ux + w2 * vx + w4 * cx + w5 * (ux * dot_vY - vx * third)
        my_ref[...] = su * uy + w2 * vy + w4 * cy + w5 * (uy * dot_vY - vy * third)
        mz_ref[...] = su * uz + w2 * vz + w4 * cz + w5 * (uz * dot_vY - vz * third)
    else:
        ms_ref[...] = w0 * s_e
        su = w1 * s_e
        mx_ref[...] = su * ux
        my_ref[...] = su * uy
        mz_ref[...] = su * uz


def _edge_compute(rb, u4, feat_e, layer, has_v):
    Ep = rb.shape[0]
    (w1, b1), (w2, b2) = layer['mlp']
    wo = layer['w_out']
    C = wo.shape[1] // 6
    F = feat_e.shape[1]
    grid = (Ep // _BE,)
    bs = lambda w: pl.BlockSpec((_BE, w), lambda i: (i, 0))
    full = lambda a: pl.BlockSpec(a.shape, lambda i: (0, 0))
    b1r = b1.reshape(1, -1)
    b2r = b2.reshape(1, -1)
    return pl.pallas_call(
        functools.partial(_edge_body, has_v),
        grid=grid,
        in_specs=[bs(8), bs(4), bs(F), full(w1), full(b1r), full(w2),
                  full(b2r), full(wo)],
        out_specs=[bs(C)] * 4,
        out_shape=[jax.ShapeDtypeStruct((Ep, C), jnp.float32)] * 4,
    )(rb, u4, feat_e, w1, b1r, w2, b2r, wo)


def _node_body(first, vscale, as_ref, ax_ref, ay_ref, az_ref, s_ref,
               vx_ref, vy_ref, vz_ref, ws_ref, wss_ref, wv_ref, wsv_ref,
               wg_ref, so_ref, xo_ref, yo_ref, zo_ref):
    inv = _INV_SQRT_AVG
    new_s = _dot(as_ref[...] * inv, ws_ref[...]) + _dot(s_ref[...], wss_ref[...])
    wv = wv_ref[...]
    nx = _dot(ax_ref[...] * inv, wv)
    ny = _dot(ay_ref[...] * inv, wv)
    nz = _dot(az_ref[...] * inv, wv)
    if not first:
        wsv = wsv_ref[...]
        nx = nx + _dot(vx_ref[...], wsv)
        ny = ny + _dot(vy_ref[...], wsv)
        nz = nz + _dot(vz_ref[...], wsv)
    gate = jax.nn.sigmoid(_dot(new_s, wg_ref[...])) * vscale
    so_ref[...] = _silu(new_s)
    xo_ref[...] = nx * gate
    yo_ref[...] = ny * gate
    zo_ref[...] = nz * gate


def _node_update(aggs, s, v3, layer, first, last):
    Np = s.shape[0]
    C = s.shape[1]
    grid = (Np // _BN,)
    bs = pl.BlockSpec((_BN, C), lambda i: (i, 0))
    full = lambda a: pl.BlockSpec(a.shape, lambda i: (0, 0))
    vscale = 0.5 if last else 1.0
    return pl.pallas_call(
        functools.partial(_node_body, first, vscale),
        grid=grid,
        in_specs=[bs] * 8 + [full(layer['W_s'])] * 5,
        out_specs=[bs] * 4,
        out_shape=[jax.ShapeDtypeStruct((Np, C), jnp.float32)] * 4,
    )(*aggs, s, *v3, layer['W_s'], layer['W_skip_s'], layer['W_v'],
      layer['W_skip_v'], layer['W_gate'])


def _gather_rows(table, idx):
    return table[idx]


def _scatter_add(msgs, recv_p, Np):
    return [jax.ops.segment_sum(m, recv_p, num_segments=Np) for m in msgs]


def kernel(positions, species, senders, receivers, params):
    N = positions.shape[0]
    E = senders.shape[0]
    C = params['embed'].shape[1]

    Np = pl.cdiv(N, 51200) * 51200
    Ep = pl.cdiv(E, 32768) * 32768

    posp = jnp.zeros((Np, 4), jnp.float32).at[:N, :3].set(positions)
    senders_p = jnp.concatenate(
        [senders, jnp.zeros((Ep - E,), senders.dtype)])
    receivers_p = jnp.concatenate(
        [receivers, jnp.full((Ep - E,), Np - 1, receivers.dtype)])

    pos_s = _gather_rows(posp, senders_p)
    pos_r = _gather_rows(posp, receivers_p)
    u4, rb = _geometry(pos_s, pos_r)

    spec_p = jnp.zeros((Np, 1), jnp.int32).at[:N, 0].set(species)
    s = _embed(spec_p, params['embed'])
    v3 = [jnp.zeros((Np, C), jnp.float32)] * 3

    layers = params['layers']
    for li, layer in enumerate(layers):
        first = li == 0
        last = li == len(layers) - 1
        feat = s if first else jnp.concatenate([s] + v3, axis=1)
        feat_e = _gather_rows(feat, senders_p)
        msgs = _edge_compute(rb, u4, feat_e, layer, has_v=not first)
        aggs = _scatter_add(msgs, receivers_p, Np)
        s, *v3 = _node_update(aggs, s, v3, layer, first, last)

    v_il = jnp.stack(v3, axis=-1).reshape(Np, 3 * C)
    out = jnp.concatenate([s, v_il], axis=1)
    return out[:N]


# speedup vs baseline: 6.9305x; 6.9305x over previous
import functools
import math

import jax
import jax.numpy as jnp
from jax.experimental import pallas as pl
from jax.experimental.pallas import tpu as pltpu

R_MAX = 5.0
_INV_SQRT_AVG = 1.0 / math.sqrt(32.0)

_BE = 2048
_BN = 2048

_PREC = jax.lax.Precision.HIGHEST


def _dot(a, b):
    return jax.lax.dot_general(a, b, (((1,), (0,)), ((), ())),
                               preferred_element_type=jnp.float32,
                               precision=_PREC)


def _silu(x):
    return x * jax.nn.sigmoid(x)


def _geom_body(ps_ref, pr_ref, u_ref, rb_ref):
    ps = ps_ref[...]
    pr = pr_ref[...]
    rel = (pr - ps) * (1.0 / R_MAX)
    r2 = jnp.sum(rel * rel, axis=1, keepdims=True)
    r = jnp.sqrt(r2)
    r_safe = jnp.maximum(r, 1e-6)
    u_ref[...] = rel / r_safe
    n = (jax.lax.broadcasted_iota(jnp.int32, (1, 8), 1) + 1).astype(jnp.float32)
    rb = math.sqrt(2.0) * jnp.sin(jnp.pi * r_safe * n) / r_safe
    rc = jnp.clip(r, 0.0, 1.0)
    rc2 = rc * rc
    rc4 = rc2 * rc2
    rc6 = rc4 * rc2
    rc7 = rc6 * rc
    rc8 = rc4 * rc4
    env = 1.0 - 28.0 * rc6 + 48.0 * rc7 - 21.0 * rc8
    rb_ref[...] = rb * env


def _geometry(pos_s, pos_r):
    Ep = pos_s.shape[0]
    grid = (Ep // _BE,)
    bs = lambda w: pl.BlockSpec((_BE, w), lambda i: (i, 0))
    return pl.pallas_call(
        _geom_body,
        grid=grid,
        in_specs=[bs(4), bs(4)],
        out_specs=[bs(4), bs(8)],
        out_shape=[jax.ShapeDtypeStruct((Ep, 4), jnp.float32),
                   jax.ShapeDtypeStruct((Ep, 8), jnp.float32)],
    )(pos_s, pos_r)


def _embed_body(spec_ref, emb_ref, out_ref):
    spec = spec_ref[...]
    ns = emb_ref.shape[0]
    oh = (spec == jax.lax.broadcasted_iota(jnp.int32, (spec.shape[0], ns), 1))
    out_ref[...] = _dot(oh.astype(jnp.float32), emb_ref[...])


def _embed(spec_p, emb):
    Np = spec_p.shape[0]
    C = emb.shape[1]
    return pl.pallas_call(
        _embed_body,
        grid=(Np // _BN,),
        in_specs=[pl.BlockSpec((_BN, 1), lambda i: (i, 0)),
                  pl.BlockSpec(emb.shape, lambda i: (0, 0))],
        out_specs=pl.BlockSpec((_BN, C), lambda i: (i, 0)),
        out_shape=jax.ShapeDtypeStruct((Np, C), jnp.float32),
    )(spec_p, emb)


def _edge_body(has_v, rb_ref, u_ref, feat_ref, w1_ref, b1_ref, w2_ref, b2_ref,
               wo_ref, ms_ref, mx_ref, my_ref, mz_ref):
    rb = rb_ref[...]
    h = _silu(_dot(rb, w1_ref[...]) + b1_ref[...])
    h = _silu(_dot(h, w2_ref[...]) + b2_ref[...])
    w = _dot(h, wo_ref[...])
    C = ms_ref.shape[1]
    w0 = w[:, 0 * C:1 * C]
    w1 = w[:, 1 * C:2 * C]
    w2 = w[:, 2 * C:3 * C]
    w3 = w[:, 3 * C:4 * C]
    w4 = w[:, 4 * C:5 * C]
    w5 = w[:, 5 * C:6 * C]
    u = u_ref[...]
    ux = u[:, 0:1]
    uy = u[:, 1:2]
    uz = u[:, 2:3]
    s_e = feat_ref[:, 0:C]
    if has_v:
        vx = feat_ref[:, C:2 * C]
        vy = feat_ref[:, 2 * C:3 * C]
        vz = feat_ref[:, 3 * C:4 * C]
        dot_vY = vx * ux + vy * uy + vz * uz
        cx = vy * uz - vz * uy
        cy = vz * ux - vx * uz
        cz = vx * uy - vy * ux
        third = 1.0 / 3.0
        ms_ref[...] = w0 * s_e + w3 * dot_vY
        su = w1 * s_e
        mx_ref[...] = su * ux + w2 * vx + w4 * cx + w5 * (ux * dot_vY - vx * third)
        my_ref[...] = su * uy + w2 * vy + w4 * cy + w5 * (uy * dot_vY - vy * third)
        mz_ref[...] = su * uz + w2 * vz + w4 * cz + w5 * (uz * dot_vY - vz * third)
    else:
        ms_ref[...] = w0 * s_e
        su = w1 * s_e
        mx_ref[...] = su * ux
        my_ref[...] = su * uy
        mz_ref[...] = su * uz


def _edge_compute(rb, u4, feat_e, layer, has_v):
    Ep = rb.shape[0]
    (w1, b1), (w2, b2) = layer['mlp']
    wo = layer['w_out']
    C = wo.shape[1] // 6
    F = feat_e.shape[1]
    grid = (Ep // _BE,)
    bs = lambda w: pl.BlockSpec((_BE, w), lambda i: (i, 0))
    full = lambda a: pl.BlockSpec(a.shape, lambda i: (0, 0))
    b1r = b1.reshape(1, -1)
    b2r = b2.reshape(1, -1)
    return pl.pallas_call(
        functools.partial(_edge_body, has_v),
        grid=grid,
        in_specs=[bs(8), bs(4), bs(F), full(w1), full(b1r), full(w2),
                  full(b2r), full(wo)],
        out_specs=[bs(C)] * 4,
        out_shape=[jax.ShapeDtypeStruct((Ep, C), jnp.float32)] * 4,
    )(rb, u4, feat_e, w1, b1r, w2, b2r, wo)


def _node_body(first, vscale, as_ref, ax_ref, ay_ref, az_ref, s_ref,
               vx_ref, vy_ref, vz_ref, ws_ref, wss_ref, wv_ref, wsv_ref,
               wg_ref, so_ref, xo_ref, yo_ref, zo_ref):
    inv = _INV_SQRT_AVG
    new_s = _dot(as_ref[...] * inv, ws_ref[...]) + _dot(s_ref[...], wss_ref[...])
    wv = wv_ref[...]
    nx = _dot(ax_ref[...] * inv, wv)
    ny = _dot(ay_ref[...] * inv, wv)
    nz = _dot(az_ref[...] * inv, wv)
    if not first:
        wsv = wsv_ref[...]
        nx = nx + _dot(vx_ref[...], wsv)
        ny = ny + _dot(vy_ref[...], wsv)
        nz = nz + _dot(vz_ref[...], wsv)
    gate = jax.nn.sigmoid(_dot(new_s, wg_ref[...])) * vscale
    so_ref[...] = _silu(new_s)
    xo_ref[...] = nx * gate
    yo_ref[...] = ny * gate
    zo_ref[...] = nz * gate


def _node_update(aggs, s, v3, layer, first, last):
    Np = s.shape[0]
    C = s.shape[1]
    grid = (Np // _BN,)
    bs = pl.BlockSpec((_BN, C), lambda i: (i, 0))
    full = lambda a: pl.BlockSpec(a.shape, lambda i: (0, 0))
    vscale = 0.5 if last else 1.0
    return pl.pallas_call(
        functools.partial(_node_body, first, vscale),
        grid=grid,
        in_specs=[bs] * 8 + [full(layer['W_s'])] * 5,
        out_specs=[bs] * 4,
        out_shape=[jax.ShapeDtypeStruct((Np, C), jnp.float32)] * 4,
    )(*aggs, s, *v3, layer['W_s'], layer['W_skip_s'], layer['W_v'],
      layer['W_skip_v'], layer['W_gate'])


def _gather_rows(table, idx):
    return table[idx]


def _scatter_add(msgs, recv_p, Np):
    return [jax.ops.segment_sum(m, recv_p, num_segments=Np) for m in msgs]


def kernel(positions, species, senders, receivers, params):
    N = positions.shape[0]
    E = senders.shape[0]
    C = params['embed'].shape[1]

    Np = pl.cdiv(N, 51200) * 51200
    Ep = pl.cdiv(E, 32768) * 32768

    posp = jnp.zeros((Np, 4), jnp.float32).at[:N, :3].set(positions)
    senders_p = jnp.concatenate(
        [senders, jnp.zeros((Ep - E,), senders.dtype)])
    receivers_p = jnp.concatenate(
        [receivers, jnp.full((Ep - E,), Np - 1, receivers.dtype)])

    pos_s = _gather_rows(posp, senders_p)
    pos_r = _gather_rows(posp, receivers_p)
    u4, rb = _geometry(pos_s, pos_r)

    spec_p = jnp.zeros((Np, 1), jnp.int32).at[:N, 0].set(species)
    s = _embed(spec_p, params['embed'])
    v3 = [jnp.zeros((Np, C), jnp.float32)] * 3

    layers = params['layers']
    for li, layer in enumerate(layers):
        first = li == 0
        last = li == len(layers) - 1
        feat = s if first else jnp.concatenate([s] + v3, axis=1)
        feat_e = _gather_rows(feat, senders_p)
        msgs = _edge_compute(rb, u4, feat_e, layer, has_v=not first)
        aggs = _scatter_add(msgs, receivers_p, Np)
        s, *v3 = _node_update(aggs, s, v3, layer, first, last)

    v_il = jnp.stack(v3, axis=-1).reshape(Np, 3 * C)
    out = jnp.concatenate([s, v_il], axis=1)
    return out[:N]


# speedup vs baseline: 18.1689x; 2.6216x over previous
import functools
import math

import jax
import jax.numpy as jnp
from jax import lax
from jax.experimental import pallas as pl
from jax.experimental.pallas import tpu as pltpu
from jax.experimental.pallas import tpu_sc as plsc

R_MAX = 5.0
_INV_SQRT_AVG = 1.0 / math.sqrt(32.0)

_BE = 2048
_BN = 2048

_NW = 32
_NT = 16

_PREC = jax.lax.Precision.HIGHEST


def _dot(a, b):
    return jax.lax.dot_general(a, b, (((1,), (0,)), ((), ())),
                               preferred_element_type=jnp.float32,
                               precision=_PREC)


def _silu(x):
    return x * jax.nn.sigmoid(x)


def _geom_body(ps_ref, pr_ref, u_ref, rb_ref):
    ps = ps_ref[...]
    pr = pr_ref[...]
    rel = (pr - ps) * (1.0 / R_MAX)
    r2 = jnp.sum(rel * rel, axis=1, keepdims=True)
    r = jnp.sqrt(r2)
    r_safe = jnp.maximum(r, 1e-6)
    u_ref[...] = rel[:, 0:4] / r_safe
    n = (jax.lax.broadcasted_iota(jnp.int32, (1, 8), 1) + 1).astype(jnp.float32)
    rb = math.sqrt(2.0) * jnp.sin(jnp.pi * r_safe * n) / r_safe
    rc = jnp.clip(r, 0.0, 1.0)
    rc2 = rc * rc
    rc4 = rc2 * rc2
    rc6 = rc4 * rc2
    rc7 = rc6 * rc
    rc8 = rc4 * rc4
    env = 1.0 - 28.0 * rc6 + 48.0 * rc7 - 21.0 * rc8
    rb_ref[...] = rb * env


def _geometry(pos_s, pos_r):
    Ep = pos_s.shape[0]
    grid = (Ep // _BE,)
    bs = lambda w: pl.BlockSpec((_BE, w), lambda i: (i, 0))
    return pl.pallas_call(
        _geom_body,
        grid=grid,
        in_specs=[bs(8), bs(8)],
        out_specs=[bs(4), bs(8)],
        out_shape=[jax.ShapeDtypeStruct((Ep, 4), jnp.float32),
                   jax.ShapeDtypeStruct((Ep, 8), jnp.float32)],
    )(pos_s, pos_r)


def _embed_body(spec_ref, emb_ref, out_ref):
    spec = spec_ref[...]
    ns = emb_ref.shape[0]
    oh = (spec == jax.lax.broadcasted_iota(jnp.int32, (spec.shape[0], ns), 1))
    out_ref[...] = _dot(oh.astype(jnp.float32), emb_ref[...])


def _embed(spec_p, emb):
    Np = spec_p.shape[0]
    C = emb.shape[1]
    return pl.pallas_call(
        _embed_body,
        grid=(Np // _BN,),
        in_specs=[pl.BlockSpec((_BN, 1), lambda i: (i, 0)),
                  pl.BlockSpec(emb.shape, lambda i: (0, 0))],
        out_specs=pl.BlockSpec((_BN, C), lambda i: (i, 0)),
        out_shape=jax.ShapeDtypeStruct((Np, C), jnp.float32),
    )(spec_p, emb)


def _edge_body(has_v, rb_ref, u_ref, feat_ref, w1_ref, b1_ref, w2_ref, b2_ref,
               wo_ref, msg_ref):
    rb = rb_ref[...]
    h = _silu(_dot(rb, w1_ref[...]) + b1_ref[...])
    h = _silu(_dot(h, w2_ref[...]) + b2_ref[...])
    w = _dot(h, wo_ref[...])
    C = msg_ref.shape[2]
    w0 = w[:, 0 * C:1 * C]
    w1 = w[:, 1 * C:2 * C]
    w2 = w[:, 2 * C:3 * C]
    w3 = w[:, 3 * C:4 * C]
    w4 = w[:, 4 * C:5 * C]
    w5 = w[:, 5 * C:6 * C]
    u = u_ref[...]
    ux = u[:, 0:1]
    uy = u[:, 1:2]
    uz = u[:, 2:3]
    s_e = feat_ref[:, 0:C]
    if has_v:
        vx = feat_ref[:, C:2 * C]
        vy = feat_ref[:, 2 * C:3 * C]
        vz = feat_ref[:, 3 * C:4 * C]
        dot_vY = vx * ux + vy * uy + vz * uz
        cx = vy * uz - vz * uy
        cy = vz * ux - vx * uz
        cz = vx * uy - vy * ux
        third = 1.0 / 3.0
        su = w1 * s_e
        msg_ref[0] = w0 * s_e + w3 * dot_vY
        msg_ref[1] = su * ux + w2 * vx + w4 * cx + w5 * (ux * dot_vY - vx * third)
        msg_ref[2] = su * uy + w2 * vy + w4 * cy + w5 * (uy * dot_vY - vy * third)
        msg_ref[3] = su * uz + w2 * vz + w4 * cz + w5 * (uz * dot_vY - vz * third)
    else:
        su = w1 * s_e
        msg_ref[0] = w0 * s_e
        msg_ref[1] = su * ux
        msg_ref[2] = su * uy
        msg_ref[3] = su * uz


def _edge_compute(rb, u4, feat_e, layer, has_v):
    Ep = rb.shape[0]
    (w1, b1), (w2, b2) = layer['mlp']
    wo = layer['w_out']
    C = wo.shape[1] // 6
    F = feat_e.shape[1]
    grid = (Ep // _BE,)
    bs = lambda w: pl.BlockSpec((_BE, w), lambda i: (i, 0))
    full = lambda a: pl.BlockSpec(a.shape, lambda i: (0, 0))
    b1r = b1.reshape(1, -1)
    b2r = b2.reshape(1, -1)
    return pl.pallas_call(
        functools.partial(_edge_body, has_v),
        grid=grid,
        in_specs=[bs(8), bs(4), bs(F), full(w1), full(b1r), full(w2),
                  full(b2r), full(wo)],
        out_specs=pl.BlockSpec((4, _BE, C), lambda i: (0, i, 0)),
        out_shape=jax.ShapeDtypeStruct((4, Ep, C), jnp.float32),
    )(rb, u4, feat_e, w1, b1r, w2, b2r, wo)


def _node_body(first, vscale, agg_ref, feat_ref, ws_ref, wss_ref, wv_ref,
               wsv_ref, wg_ref, fo_ref):
    inv = _INV_SQRT_AVG
    C = ws_ref.shape[0]
    s = feat_ref[:, 0:C]
    new_s = _dot(agg_ref[0] * inv, ws_ref[...]) + _dot(s, wss_ref[...])
    wv = wv_ref[...]
    nx = _dot(agg_ref[1] * inv, wv)
    ny = _dot(agg_ref[2] * inv, wv)
    nz = _dot(agg_ref[3] * inv, wv)
    if not first:
        wsv = wsv_ref[...]
        nx = nx + _dot(feat_ref[:, C:2 * C], wsv)
        ny = ny + _dot(feat_ref[:, 2 * C:3 * C], wsv)
        nz = nz + _dot(feat_ref[:, 3 * C:4 * C], wsv)
    gate = jax.nn.sigmoid(_dot(new_s, wg_ref[...])) * vscale
    fo_ref[:, 0:C] = _silu(new_s)
    fo_ref[:, C:2 * C] = nx * gate
    fo_ref[:, 2 * C:3 * C] = ny * gate
    fo_ref[:, 3 * C:4 * C] = nz * gate


def _node_update(agg, feat, layer, first, last):
    Np = feat.shape[0]
    C = layer['W_s'].shape[0]
    grid = (Np // _BN,)
    full = lambda a: pl.BlockSpec(a.shape, lambda i: (0, 0))
    vscale = 0.5 if last else 1.0
    return pl.pallas_call(
        functools.partial(_node_body, first, vscale),
        grid=grid,
        in_specs=[pl.BlockSpec((4, _BN, C), lambda i: (0, i, 0)),
                  pl.BlockSpec((_BN, feat.shape[1]), lambda i: (i, 0))]
                 + [full(layer['W_s'])] * 5,
        out_specs=pl.BlockSpec((_BN, 4 * C), lambda i: (i, 0)),
        out_shape=jax.ShapeDtypeStruct((Np, 4 * C), jnp.float32),
    )(agg, feat, layer['W_s'], layer['W_skip_s'], layer['W_v'],
      layer['W_skip_v'], layer['W_gate'])


def _sc_gather(table, idx):
    Np, F = table.shape
    Ep = idx.shape[0]
    Ew = Ep // _NW
    G = 512
    nchunk = Ew // G
    mesh = plsc.VectorSubcoreMesh(core_axis_name="c", subcore_axis_name="s")

    @functools.partial(
        pl.kernel, mesh=mesh,
        out_type=jax.ShapeDtypeStruct((Ep, F), jnp.float32),
        compiler_params=pltpu.CompilerParams(use_tc_tiling_on_sc=False),
        scratch_types=[pltpu.VMEM((Ew,), jnp.int32),
                       pltpu.VMEM((G, F), jnp.float32),
                       pltpu.SemaphoreType.DMA],
    )
    def k(table_hbm, idx_hbm, out_hbm, idx_v, rows_v, sem):
        wid = lax.axis_index("s") * 2 + lax.axis_index("c")
        base = wid * Ew
        pltpu.sync_copy(idx_hbm.at[pl.ds(base, Ew)], idx_v)

        def body(j, carry):
            off = j * G
            descs = []
            for g in range(G // 128):
                descs.append(pltpu.async_copy(
                    table_hbm.at[idx_v.at[pl.ds(off + g * 128, 128)]],
                    rows_v.at[pl.ds(g * 128, 128)], sem))
            for d in descs:
                d.wait()
            pltpu.sync_copy(rows_v, out_hbm.at[pl.ds(base + off, G)])
            return carry

        lax.fori_loop(0, nchunk, body, 0)

    return k(table, idx)


def _sc_scatter(msg, recv2d, zeros_t, Np):
    _, Ep, C = msg.shape
    Et = Ep // _NT
    G = 512
    nchunk = Et // G
    Rt = Np // _NT
    mesh = plsc.VectorSubcoreMesh(core_axis_name="c", subcore_axis_name="s")

    @functools.partial(
        pl.kernel, mesh=mesh,
        out_type=jax.ShapeDtypeStruct((4, Np, C), jnp.float32),
        compiler_params=pltpu.CompilerParams(use_tc_tiling_on_sc=False),
        scratch_types=[pltpu.VMEM((G // 128, 128), jnp.int32),
                       pltpu.VMEM((G, C), jnp.float32),
                       pltpu.VMEM_SHARED((Np, C), jnp.float32),
                       pltpu.SemaphoreType.DMA],
    )
    def k(msg_hbm, recv_hbm, zeros_hbm, out_hbm, idx_v, rows_v, acc_sh, sem):
        cid = lax.axis_index("c")
        sid = lax.axis_index("s")
        for t in range(2):
            chunk = cid * 2 + t
            pltpu.sync_copy(zeros_hbm, acc_sh.at[pl.ds(sid * Rt, Rt)])
            plsc.subcore_barrier()

            def body(j, carry):
                off = sid * Et + j * G
                row_off = sid * (Et // 128) + j * (G // 128)
                pltpu.sync_copy(recv_hbm.at[pl.ds(row_off, G // 128)], idx_v)
                pltpu.sync_copy(msg_hbm.at[chunk, pl.ds(off, G)], rows_v)
                for g in range(G // 128):
                    pltpu.sync_copy(rows_v.at[pl.ds(g * 128, 128)],
                                    acc_sh.at[idx_v.at[g]], add=True)
                return carry

            lax.fori_loop(0, nchunk, body, 0)
            plsc.subcore_barrier()
            pltpu.sync_copy(acc_sh.at[pl.ds(sid * Rt, Rt)],
                            out_hbm.at[chunk, pl.ds(sid * Rt, Rt)])
            plsc.subcore_barrier()

    return k(msg, recv2d, zeros_t)


def kernel(positions, species, senders, receivers, params):
    N = positions.shape[0]
    E = senders.shape[0]
    C = params['embed'].shape[1]

    Np = pl.cdiv(N, 51200) * 51200
    Ep = pl.cdiv(E, 32768) * 32768

    posp = jnp.zeros((Np, 8), jnp.float32).at[:N, :3].set(positions)
    senders_p = jnp.concatenate(
        [senders, jnp.zeros((Ep - E,), senders.dtype)])
    receivers_p = jnp.concatenate(
        [receivers, jnp.full((Ep - E,), Np - 1, receivers.dtype)])
    recv2d = receivers_p.reshape(Ep // 128, 128)
    zeros_t = jnp.zeros((Np // _NT, C), jnp.float32)

    pos_s = _sc_gather(posp, senders_p)
    pos_r = _sc_gather(posp, receivers_p)
    u4, rb = _geometry(pos_s, pos_r)

    spec_p = jnp.zeros((Np, 1), jnp.int32).at[:N, 0].set(species)
    feat = _embed(spec_p, params['embed'])

    layers = params['layers']
    for li, layer in enumerate(layers):
        first = li == 0
        last = li == len(layers) - 1
        feat_e = _sc_gather(feat, senders_p)
        msg = _edge_compute(rb, u4, feat_e, layer, has_v=not first)
        agg = _sc_scatter(msg, recv2d, zeros_t, Np)
        feat = _node_update(agg, feat, layer, first, last)

    s = feat[:, 0:C]
    v_il = jnp.stack([feat[:, C:2 * C], feat[:, 2 * C:3 * C],
                      feat[:, 3 * C:4 * C]], axis=-1).reshape(Np, 3 * C)
    out = jnp.concatenate([s, v_il], axis=1)
    return out[:N]


# speedup vs baseline: 21.2074x; 1.1672x over previous
import functools
import math

import jax
import jax.numpy as jnp
from jax import lax
from jax.experimental import pallas as pl
from jax.experimental.pallas import tpu as pltpu
from jax.experimental.pallas import tpu_sc as plsc

R_MAX = 5.0
_INV_SQRT_AVG = 1.0 / math.sqrt(32.0)

_BE = 2048
_BN = 2048

_NW = 32
_NT = 16

_PREC = jax.lax.Precision.DEFAULT


def _dot(a, b):
    return jax.lax.dot_general(a, b, (((1,), (0,)), ((), ())),
                               preferred_element_type=jnp.float32,
                               precision=_PREC)


def _silu(x):
    return x * jax.nn.sigmoid(x)


def _geom_body(ps_ref, pr_ref, u_ref, rb_ref):
    ps = ps_ref[...]
    pr = pr_ref[...]
    rel = (pr - ps) * (1.0 / R_MAX)
    r2 = jnp.sum(rel * rel, axis=1, keepdims=True)
    r = jnp.sqrt(r2)
    r_safe = jnp.maximum(r, 1e-6)
    u_ref[...] = rel[:, 0:4] / r_safe
    n = (jax.lax.broadcasted_iota(jnp.int32, (1, 8), 1) + 1).astype(jnp.float32)
    rb = math.sqrt(2.0) * jnp.sin(jnp.pi * r_safe * n) / r_safe
    rc = jnp.clip(r, 0.0, 1.0)
    rc2 = rc * rc
    rc4 = rc2 * rc2
    rc6 = rc4 * rc2
    rc7 = rc6 * rc
    rc8 = rc4 * rc4
    env = 1.0 - 28.0 * rc6 + 48.0 * rc7 - 21.0 * rc8
    rb_ref[...] = rb * env


def _geometry(pos_s, pos_r):
    Ep = pos_s.shape[0]
    grid = (Ep // _BE,)
    bs = lambda w: pl.BlockSpec((_BE, w), lambda i: (i, 0))
    return pl.pallas_call(
        _geom_body,
        grid=grid,
        in_specs=[bs(8), bs(8)],
        out_specs=[bs(4), bs(8)],
        out_shape=[jax.ShapeDtypeStruct((Ep, 4), jnp.float32),
                   jax.ShapeDtypeStruct((Ep, 8), jnp.float32)],
    )(pos_s, pos_r)


def _embed_body(spec_ref, emb_ref, out_ref):
    spec = spec_ref[...]
    ns = emb_ref.shape[0]
    oh = (spec == jax.lax.broadcasted_iota(jnp.int32, (spec.shape[0], ns), 1))
    out_ref[...] = _dot(oh.astype(jnp.float32), emb_ref[...])


def _embed(spec_p, emb):
    Np = spec_p.shape[0]
    C = emb.shape[1]
    return pl.pallas_call(
        _embed_body,
        grid=(Np // _BN,),
        in_specs=[pl.BlockSpec((_BN, 1), lambda i: (i, 0)),
                  pl.BlockSpec(emb.shape, lambda i: (0, 0))],
        out_specs=pl.BlockSpec((_BN, C), lambda i: (i, 0)),
        out_shape=jax.ShapeDtypeStruct((Np, C), jnp.float32),
    )(spec_p, emb)


def _edge_body(has_v, rb_ref, u_ref, feat_ref, w1_ref, b1_ref, w2_ref, b2_ref,
               wo_ref, msg_ref):
    rb = rb_ref[...]
    h = _silu(_dot(rb, w1_ref[...]) + b1_ref[...])
    h = _silu(_dot(h, w2_ref[...]) + b2_ref[...])
    w = _dot(h, wo_ref[...])
    C = msg_ref.shape[2]
    w0 = w[:, 0 * C:1 * C]
    w1 = w[:, 1 * C:2 * C]
    w2 = w[:, 2 * C:3 * C]
    w3 = w[:, 3 * C:4 * C]
    w4 = w[:, 4 * C:5 * C]
    w5 = w[:, 5 * C:6 * C]
    u = u_ref[...]
    ux = u[:, 0:1]
    uy = u[:, 1:2]
    uz = u[:, 2:3]
    s_e = feat_ref[:, 0:C]
    if has_v:
        vx = feat_ref[:, C:2 * C]
        vy = feat_ref[:, 2 * C:3 * C]
        vz = feat_ref[:, 3 * C:4 * C]
        dot_vY = vx * ux + vy * uy + vz * uz
        cx = vy * uz - vz * uy
        cy = vz * ux - vx * uz
        cz = vx * uy - vy * ux
        third = 1.0 / 3.0
        su = w1 * s_e
        msg_ref[0] = w0 * s_e + w3 * dot_vY
        msg_ref[1] = su * ux + w2 * vx + w4 * cx + w5 * (ux * dot_vY - vx * third)
        msg_ref[2] = su * uy + w2 * vy + w4 * cy + w5 * (uy * dot_vY - vy * third)
        msg_ref[3] = su * uz + w2 * vz + w4 * cz + w5 * (uz * dot_vY - vz * third)
    else:
        su = w1 * s_e
        msg_ref[0] = w0 * s_e
        msg_ref[1] = su * ux
        msg_ref[2] = su * uy
        msg_ref[3] = su * uz


def _edge_compute(rb, u4, feat_e, layer, has_v):
    Ep = rb.shape[0]
    (w1, b1), (w2, b2) = layer['mlp']
    wo = layer['w_out']
    C = wo.shape[1] // 6
    F = feat_e.shape[1]
    grid = (Ep // _BE,)
    bs = lambda w: pl.BlockSpec((_BE, w), lambda i: (i, 0))
    full = lambda a: pl.BlockSpec(a.shape, lambda i: (0, 0))
    b1r = b1.reshape(1, -1)
    b2r = b2.reshape(1, -1)
    return pl.pallas_call(
        functools.partial(_edge_body, has_v),
        grid=grid,
        in_specs=[bs(8), bs(4), bs(F), full(w1), full(b1r), full(w2),
                  full(b2r), full(wo)],
        out_specs=pl.BlockSpec((4, _BE, C), lambda i: (0, i, 0)),
        out_shape=jax.ShapeDtypeStruct((4, Ep, C), jnp.float32),
    )(rb, u4, feat_e, w1, b1r, w2, b2r, wo)


def _node_body(first, vscale, agg_ref, feat_ref, ws_ref, wss_ref, wv_ref,
               wsv_ref, wg_ref, fo_ref):
    inv = _INV_SQRT_AVG
    C = ws_ref.shape[0]
    s = feat_ref[:, 0:C]
    new_s = _dot(agg_ref[0] * inv, ws_ref[...]) + _dot(s, wss_ref[...])
    wv = wv_ref[...]
    nx = _dot(agg_ref[1] * inv, wv)
    ny = _dot(agg_ref[2] * inv, wv)
    nz = _dot(agg_ref[3] * inv, wv)
    if not first:
        wsv = wsv_ref[...]
        nx = nx + _dot(feat_ref[:, C:2 * C], wsv)
        ny = ny + _dot(feat_ref[:, 2 * C:3 * C], wsv)
        nz = nz + _dot(feat_ref[:, 3 * C:4 * C], wsv)
    gate = jax.nn.sigmoid(_dot(new_s, wg_ref[...])) * vscale
    fo_ref[:, 0:C] = _silu(new_s)
    fo_ref[:, C:2 * C] = nx * gate
    fo_ref[:, 2 * C:3 * C] = ny * gate
    fo_ref[:, 3 * C:4 * C] = nz * gate


def _node_update(agg, feat, layer, first, last):
    Np = feat.shape[0]
    C = layer['W_s'].shape[0]
    grid = (Np // _BN,)
    full = lambda a: pl.BlockSpec(a.shape, lambda i: (0, 0))
    vscale = 0.5 if last else 1.0
    return pl.pallas_call(
        functools.partial(_node_body, first, vscale),
        grid=grid,
        in_specs=[pl.BlockSpec((4, _BN, C), lambda i: (0, i, 0)),
                  pl.BlockSpec((_BN, feat.shape[1]), lambda i: (i, 0))]
                 + [full(layer['W_s'])] * 5,
        out_specs=pl.BlockSpec((_BN, 4 * C), lambda i: (i, 0)),
        out_shape=jax.ShapeDtypeStruct((Np, 4 * C), jnp.float32),
    )(agg, feat, layer['W_s'], layer['W_skip_s'], layer['W_v'],
      layer['W_skip_v'], layer['W_gate'])


def _sc_gather(table, idx):
    Np, F = table.shape
    Ep = idx.shape[0]
    Ew = Ep // _NW
    G = 512
    nchunk = Ew // G
    mesh = plsc.VectorSubcoreMesh(core_axis_name="c", subcore_axis_name="s")

    @functools.partial(
        pl.kernel, mesh=mesh,
        out_type=jax.ShapeDtypeStruct((Ep, F), jnp.float32),
        compiler_params=pltpu.CompilerParams(use_tc_tiling_on_sc=False),
        scratch_types=[pltpu.VMEM((Ew,), jnp.int32),
                       pltpu.VMEM((G, F), jnp.float32),
                       pltpu.SemaphoreType.DMA],
    )
    def k(table_hbm, idx_hbm, out_hbm, idx_v, rows_v, sem):
        wid = lax.axis_index("s") * 2 + lax.axis_index("c")
        base = wid * Ew
        pltpu.sync_copy(idx_hbm.at[pl.ds(base, Ew)], idx_v)

        def body(j, carry):
            off = j * G
            descs = []
            for g in range(G // 128):
                descs.append(pltpu.async_copy(
                    table_hbm.at[idx_v.at[pl.ds(off + g * 128, 128)]],
                    rows_v.at[pl.ds(g * 128, 128)], sem))
            for d in descs:
                d.wait()
            pltpu.sync_copy(rows_v, out_hbm.at[pl.ds(base + off, G)])
            return carry

        lax.fori_loop(0, nchunk, body, 0)

    return k(table, idx)


def _sc_scatter(msg, recv2d, zeros_t, Np):
    _, Ep, C = msg.shape
    Et = Ep // _NT
    G = 512
    nchunk = Et // G
    Rt = Np // _NT
    mesh = plsc.VectorSubcoreMesh(core_axis_name="c", subcore_axis_name="s")

    @functools.partial(
        pl.kernel, mesh=mesh,
        out_type=jax.ShapeDtypeStruct((4, Np, C), jnp.float32),
        compiler_params=pltpu.CompilerParams(use_tc_tiling_on_sc=False),
        scratch_types=[pltpu.VMEM((G // 128, 128), jnp.int32),
                       pltpu.VMEM((G, C), jnp.float32),
                       pltpu.VMEM_SHARED((Np, C), jnp.float32),
                       pltpu.SemaphoreType.DMA],
    )
    def k(msg_hbm, recv_hbm, zeros_hbm, out_hbm, idx_v, rows_v, acc_sh, sem):
        cid = lax.axis_index("c")
        sid = lax.axis_index("s")
        for t in range(2):
            chunk = cid * 2 + t
            pltpu.sync_copy(zeros_hbm, acc_sh.at[pl.ds(sid * Rt, Rt)])
            plsc.subcore_barrier()

            def body(j, carry):
                off = sid * Et + j * G
                row_off = sid * (Et // 128) + j * (G // 128)
                pltpu.sync_copy(recv_hbm.at[pl.ds(row_off, G // 128)], idx_v)
                pltpu.sync_copy(msg_hbm.at[chunk, pl.ds(off, G)], rows_v)
                for g in range(G // 128):
                    pltpu.sync_copy(rows_v.at[pl.ds(g * 128, 128)],
                                    acc_sh.at[idx_v.at[g]], add=True)
                return carry

            lax.fori_loop(0, nchunk, body, 0)
            plsc.subcore_barrier()
            pltpu.sync_copy(acc_sh.at[pl.ds(sid * Rt, Rt)],
                            out_hbm.at[chunk, pl.ds(sid * Rt, Rt)])
            plsc.subcore_barrier()

    return k(msg, recv2d, zeros_t)


def kernel(positions, species, senders, receivers, params):
    N = positions.shape[0]
    E = senders.shape[0]
    C = params['embed'].shape[1]

    Np = pl.cdiv(N, 51200) * 51200
    Ep = pl.cdiv(E, 32768) * 32768

    posp = jnp.zeros((Np, 8), jnp.float32).at[:N, :3].set(positions)
    senders_p = jnp.concatenate(
        [senders, jnp.zeros((Ep - E,), senders.dtype)])
    receivers_p = jnp.concatenate(
        [receivers, jnp.full((Ep - E,), Np - 1, receivers.dtype)])
    recv2d = receivers_p.reshape(Ep // 128, 128)
    zeros_t = jnp.zeros((Np // _NT, C), jnp.float32)

    pos_s = _sc_gather(posp, senders_p)
    pos_r = _sc_gather(posp, receivers_p)
    u4, rb = _geometry(pos_s, pos_r)

    spec_p = jnp.zeros((Np, 1), jnp.int32).at[:N, 0].set(species)
    feat = _embed(spec_p, params['embed'])

    layers = params['layers']
    for li, layer in enumerate(layers):
        first = li == 0
        last = li == len(layers) - 1
        feat_e = _sc_gather(feat, senders_p)
        msg = _edge_compute(rb, u4, feat_e, layer, has_v=not first)
        agg = _sc_scatter(msg, recv2d, zeros_t, Np)
        feat = _node_update(agg, feat, layer, first, last)

    s = feat[:, 0:C]
    v_il = jnp.stack([feat[:, C:2 * C], feat[:, 2 * C:3 * C],
                      feat[:, 3 * C:4 * C]], axis=-1).reshape(Np, 3 * C)
    out = jnp.concatenate([s, v_il], axis=1)
    return out[:N]


# speedup vs baseline: 23.8510x; 1.1247x over previous
import functools
import math

import jax
import jax.numpy as jnp
from jax import lax
from jax.experimental import pallas as pl
from jax.experimental.pallas import tpu as pltpu
from jax.experimental.pallas import tpu_sc as plsc

R_MAX = 5.0
_INV_SQRT_AVG = 1.0 / math.sqrt(32.0)

_BE = 2048
_BN = 2048

_NW = 32
_NT = 16

_PREC = jax.lax.Precision.DEFAULT


def _dot(a, b):
    return jax.lax.dot_general(a, b, (((1,), (0,)), ((), ())),
                               preferred_element_type=jnp.float32,
                               precision=_PREC)


def _silu(x):
    return x * jax.nn.sigmoid(x)


def _geom_body(ps_ref, pr_ref, u_ref, rb_ref):
    ps = ps_ref[...]
    pr = pr_ref[...]
    rel = (pr - ps) * (1.0 / R_MAX)
    r2 = jnp.sum(rel * rel, axis=1, keepdims=True)
    r = jnp.sqrt(r2)
    r_safe = jnp.maximum(r, 1e-6)
    u_ref[...] = rel[:, 0:4] / r_safe
    n = (jax.lax.broadcasted_iota(jnp.int32, (1, 8), 1) + 1).astype(jnp.float32)
    rb = math.sqrt(2.0) * jnp.sin(jnp.pi * r_safe * n) / r_safe
    rc = jnp.clip(r, 0.0, 1.0)
    rc2 = rc * rc
    rc4 = rc2 * rc2
    rc6 = rc4 * rc2
    rc7 = rc6 * rc
    rc8 = rc4 * rc4
    env = 1.0 - 28.0 * rc6 + 48.0 * rc7 - 21.0 * rc8
    rb_ref[...] = rb * env


def _geometry(pos_s, pos_r):
    Ep = pos_s.shape[0]
    grid = (Ep // _BE,)
    bs = lambda w: pl.BlockSpec((_BE, w), lambda i: (i, 0))
    return pl.pallas_call(
        _geom_body,
        grid=grid,
        in_specs=[bs(8), bs(8)],
        out_specs=[bs(4), bs(8)],
        out_shape=[jax.ShapeDtypeStruct((Ep, 4), jnp.float32),
                   jax.ShapeDtypeStruct((Ep, 8), jnp.float32)],
    )(pos_s, pos_r)


def _embed_body(spec_ref, emb_ref, out_ref):
    spec = spec_ref[...]
    ns = emb_ref.shape[0]
    oh = (spec == jax.lax.broadcasted_iota(jnp.int32, (spec.shape[0], ns), 1))
    out_ref[...] = _dot(oh.astype(jnp.float32), emb_ref[...])


def _embed(spec_p, emb):
    Np = spec_p.shape[0]
    C = emb.shape[1]
    return pl.pallas_call(
        _embed_body,
        grid=(Np // _BN,),
        in_specs=[pl.BlockSpec((_BN, 1), lambda i: (i, 0)),
                  pl.BlockSpec(emb.shape, lambda i: (0, 0))],
        out_specs=pl.BlockSpec((_BN, C), lambda i: (i, 0)),
        out_shape=jax.ShapeDtypeStruct((Np, C), jnp.float32),
    )(spec_p, emb)


def _edge_body(has_v, rb_ref, u_ref, feat_ref, w1_ref, b1_ref, w2_ref, b2_ref,
               wo_ref, ub_ref, sel_ref, msg_ref):
    rb = rb_ref[...]
    h = _silu(_dot(rb, w1_ref[...]) + b1_ref[...])
    h = _silu(_dot(h, w2_ref[...]) + b2_ref[...])
    w0 = _dot(h, wo_ref[0])
    w1 = _dot(h, wo_ref[1])
    u = u_ref[...]
    ux = _dot(u, ub_ref[0])
    uy = _dot(u, ub_ref[1])
    uz = _dot(u, ub_ref[2])
    if has_v:
        F = feat_ref[...]
        s_e = _dot(F, sel_ref[0])
        vx = _dot(F, sel_ref[1])
        vy = _dot(F, sel_ref[2])
        vz = _dot(F, sel_ref[3])
        w2c = _dot(h, wo_ref[2])
        w3 = _dot(h, wo_ref[3])
        w4 = _dot(h, wo_ref[4])
        w5 = _dot(h, wo_ref[5])
        dot_vY = vx * ux + vy * uy + vz * uz
        cx = vy * uz - vz * uy
        cy = vz * ux - vx * uz
        cz = vx * uy - vy * ux
        su = w1 * s_e
        w2m = w2c - w5 * (1.0 / 3.0)
        msg_ref[0] = w0 * s_e + w3 * dot_vY
        msg_ref[1] = su * ux + w2m * vx + w4 * cx + w5 * (ux * dot_vY)
        msg_ref[2] = su * uy + w2m * vy + w4 * cy + w5 * (uy * dot_vY)
        msg_ref[3] = su * uz + w2m * vz + w4 * cz + w5 * (uz * dot_vY)
    else:
        s_e = feat_ref[...]
        su = w1 * s_e
        msg_ref[0] = w0 * s_e
        msg_ref[1] = su * ux
        msg_ref[2] = su * uy
        msg_ref[3] = su * uz


def _edge_compute(rb, u4, feat_e, layer, has_v):
    Ep = rb.shape[0]
    (w1, b1), (w2, b2) = layer['mlp']
    wo = layer['w_out']
    C = wo.shape[1] // 6
    F = feat_e.shape[1]
    grid = (Ep // _BE,)
    bs = lambda w: pl.BlockSpec((_BE, w), lambda i: (i, 0))
    full = lambda a: pl.BlockSpec(a.shape, lambda i: (0,) * a.ndim)
    b1r = b1.reshape(1, -1)
    b2r = b2.reshape(1, -1)
    wo6 = wo.T.reshape(6, C, -1).transpose(0, 2, 1)
    ub = jnp.zeros((3, 4, C), jnp.float32)
    ub = ub.at[0, 0].set(1.0).at[1, 1].set(1.0).at[2, 2].set(1.0)
    eye = jnp.eye(C, dtype=jnp.float32)
    sel = jnp.stack([jnp.zeros((4 * C, C), jnp.float32).at[k * C:(k + 1) * C].set(eye)
                     for k in range(4)])
    return pl.pallas_call(
        functools.partial(_edge_body, has_v),
        grid=grid,
        in_specs=[bs(8), bs(4), bs(F), full(w1), full(b1r), full(w2),
                  full(b2r), full(wo6), full(ub), full(sel)],
        out_specs=pl.BlockSpec((4, _BE, C), lambda i: (0, i, 0)),
        out_shape=jax.ShapeDtypeStruct((4, Ep, C), jnp.float32),
    )(rb, u4, feat_e, w1, b1r, w2, b2r, wo6, ub, sel)


def _node_body(first, vscale, agg_ref, feat_ref, ws_ref, wss_ref, wv_ref,
               wsv_ref, wg_ref, fo_ref):
    inv = _INV_SQRT_AVG
    C = ws_ref.shape[0]
    s = feat_ref[:, 0:C]
    new_s = _dot(agg_ref[0] * inv, ws_ref[...]) + _dot(s, wss_ref[...])
    wv = wv_ref[...]
    nx = _dot(agg_ref[1] * inv, wv)
    ny = _dot(agg_ref[2] * inv, wv)
    nz = _dot(agg_ref[3] * inv, wv)
    if not first:
        wsv = wsv_ref[...]
        nx = nx + _dot(feat_ref[:, C:2 * C], wsv)
        ny = ny + _dot(feat_ref[:, 2 * C:3 * C], wsv)
        nz = nz + _dot(feat_ref[:, 3 * C:4 * C], wsv)
    gate = jax.nn.sigmoid(_dot(new_s, wg_ref[...])) * vscale
    fo_ref[:, 0:C] = _silu(new_s)
    fo_ref[:, C:2 * C] = nx * gate
    fo_ref[:, 2 * C:3 * C] = ny * gate
    fo_ref[:, 3 * C:4 * C] = nz * gate


def _node_update(agg, feat, layer, first, last):
    Np = feat.shape[0]
    C = layer['W_s'].shape[0]
    grid = (Np // _BN,)
    full = lambda a: pl.BlockSpec(a.shape, lambda i: (0, 0))
    vscale = 0.5 if last else 1.0
    return pl.pallas_call(
        functools.partial(_node_body, first, vscale),
        grid=grid,
        in_specs=[pl.BlockSpec((4, _BN, C), lambda i: (0, i, 0)),
                  pl.BlockSpec((_BN, feat.shape[1]), lambda i: (i, 0))]
                 + [full(layer['W_s'])] * 5,
        out_specs=pl.BlockSpec((_BN, 4 * C), lambda i: (i, 0)),
        out_shape=jax.ShapeDtypeStruct((Np, 4 * C), jnp.float32),
    )(agg, feat, layer['W_s'], layer['W_skip_s'], layer['W_v'],
      layer['W_skip_v'], layer['W_gate'])


def _sc_gather(table, idx):
    Np, F = table.shape
    Ep = idx.shape[0]
    Ew = Ep // _NW
    G = 512
    nchunk = Ew // G
    mesh = plsc.VectorSubcoreMesh(core_axis_name="c", subcore_axis_name="s")

    @functools.partial(
        pl.kernel, mesh=mesh,
        out_type=jax.ShapeDtypeStruct((Ep, F), jnp.float32),
        compiler_params=pltpu.CompilerParams(use_tc_tiling_on_sc=False),
        scratch_types=[pltpu.VMEM((Ew,), jnp.int32),
                       pltpu.VMEM((G, F), jnp.float32),
                       pltpu.SemaphoreType.DMA],
    )
    def k(table_hbm, idx_hbm, out_hbm, idx_v, rows_v, sem):
        wid = lax.axis_index("s") * 2 + lax.axis_index("c")
        base = wid * Ew
        pltpu.sync_copy(idx_hbm.at[pl.ds(base, Ew)], idx_v)

        def body(j, carry):
            off = j * G
            descs = []
            for g in range(G // 128):
                descs.append(pltpu.async_copy(
                    table_hbm.at[idx_v.at[pl.ds(off + g * 128, 128)]],
                    rows_v.at[pl.ds(g * 128, 128)], sem))
            for d in descs:
                d.wait()
            pltpu.sync_copy(rows_v, out_hbm.at[pl.ds(base + off, G)])
            return carry

        lax.fori_loop(0, nchunk, body, 0)

    return k(table, idx)


def _sc_scatter(msg, recv2d, zeros_t, Np):
    _, Ep, C = msg.shape
    Et = Ep // _NT
    G = 512
    nchunk = Et // G
    Rt = Np // _NT
    mesh = plsc.VectorSubcoreMesh(core_axis_name="c", subcore_axis_name="s")

    @functools.partial(
        pl.kernel, mesh=mesh,
        out_type=jax.ShapeDtypeStruct((4, Np, C), jnp.float32),
        compiler_params=pltpu.CompilerParams(use_tc_tiling_on_sc=False),
        scratch_types=[pltpu.VMEM((G // 128, 128), jnp.int32),
                       pltpu.VMEM((G, C), jnp.float32),
                       pltpu.VMEM_SHARED((Np, C), jnp.float32),
                       pltpu.SemaphoreType.DMA],
    )
    def k(msg_hbm, recv_hbm, zeros_hbm, out_hbm, idx_v, rows_v, acc_sh, sem):
        cid = lax.axis_index("c")
        sid = lax.axis_index("s")
        for t in range(2):
            chunk = cid * 2 + t
            pltpu.sync_copy(zeros_hbm, acc_sh.at[pl.ds(sid * Rt, Rt)])
            plsc.subcore_barrier()

            def body(j, carry):
                off = sid * Et + j * G
                row_off = sid * (Et // 128) + j * (G // 128)
                pltpu.sync_copy(recv_hbm.at[pl.ds(row_off, G // 128)], idx_v)
                pltpu.sync_copy(msg_hbm.at[chunk, pl.ds(off, G)], rows_v)
                for g in range(G // 128):
                    pltpu.sync_copy(rows_v.at[pl.ds(g * 128, 128)],
                                    acc_sh.at[idx_v.at[g]], add=True)
                return carry

            lax.fori_loop(0, nchunk, body, 0)
            plsc.subcore_barrier()
            pltpu.sync_copy(acc_sh.at[pl.ds(sid * Rt, Rt)],
                            out_hbm.at[chunk, pl.ds(sid * Rt, Rt)])
            plsc.subcore_barrier()

    return k(msg, recv2d, zeros_t)


def kernel(positions, species, senders, receivers, params):
    N = positions.shape[0]
    E = senders.shape[0]
    C = params['embed'].shape[1]

    Np = pl.cdiv(N, 51200) * 51200
    Ep = pl.cdiv(E, 32768) * 32768

    posp = jnp.zeros((Np, 8), jnp.float32).at[:N, :3].set(positions)
    senders_p = jnp.concatenate(
        [senders, jnp.zeros((Ep - E,), senders.dtype)])
    receivers_p = jnp.concatenate(
        [receivers, jnp.full((Ep - E,), Np - 1, receivers.dtype)])
    recv2d = receivers_p.reshape(Ep // 128, 128)
    zeros_t = jnp.zeros((Np // _NT, C), jnp.float32)

    pos_s = _sc_gather(posp, senders_p)
    pos_r = _sc_gather(posp, receivers_p)
    u4, rb = _geometry(pos_s, pos_r)

    spec_p = jnp.zeros((Np, 1), jnp.int32).at[:N, 0].set(species)
    feat = _embed(spec_p, params['embed'])

    layers = params['layers']
    for li, layer in enumerate(layers):
        first = li == 0
        last = li == len(layers) - 1
        feat_e = _sc_gather(feat, senders_p)
        msg = _edge_compute(rb, u4, feat_e, layer, has_v=not first)
        agg = _sc_scatter(msg, recv2d, zeros_t, Np)
        feat = _node_update(agg, feat, layer, first, last)

    s = feat[:, 0:C]
    v_il = jnp.stack([feat[:, C:2 * C], feat[:, 2 * C:3 * C],
                      feat[:, 3 * C:4 * C]], axis=-1).reshape(Np, 3 * C)
    out = jnp.concatenate([s, v_il], axis=1)
    return out[:N]
